# Initial kernel scaffold; baseline (speedup 1.0000x reference)
import jax
import jax.numpy as jnp
from jax.experimental import pallas as pl


def kernel(x, edge_index, pred_edges, W1l, b1, W1r, W2l, b2, W2r):
    raise NotImplementedError("write your pallas kernel here")


# speedup vs baseline: 2.7744x; 2.7744x over previous
import jax
import jax.numpy as jnp
from jax import lax
from jax.experimental import pallas as pl
from jax.experimental.pallas import tpu as pltpu
from jax.experimental.pallas import tpu_sc as plsc

NC = 2
NS = 16
NW = NC * NS
LANES = 16
CHUNK = 128


def _mesh():
  return plsc.VectorSubcoreMesh(core_axis_name="c", subcore_axis_name="s",
                                num_cores=NC, num_subcores=NS)


def _agg(xh, src, dst, with_deg):
  _, n, hd = xh.shape
  ep = src.shape[0]
  T = ep // (NS * CHUNK)
  orow = (-(-n // NS) + 7) // 8 * 8
  npad = orow * NS

  outs = [jax.ShapeDtypeStruct((NC, npad, hd), jnp.float32)]
  scratch = [
      pltpu.VMEM((CHUNK,), jnp.int32),
      pltpu.VMEM((CHUNK,), jnp.int32),
      pltpu.VMEM((CHUNK, hd), jnp.float32),
      pltpu.VMEM_SHARED((npad, hd), jnp.float32),
      pltpu.SemaphoreType.DMA,
  ]
  if with_deg:
    outs.append(jax.ShapeDtypeStruct((npad, LANES), jnp.float32))
    scratch += [
        pltpu.VMEM((CHUNK, LANES), jnp.float32),
        pltpu.VMEM((CHUNK, LANES), jnp.float32),
        pltpu.VMEM_SHARED((npad, LANES), jnp.float32),
    ]

  def body(x_hbm, src_hbm, dst_hbm, *rest):
    if with_deg:
      (agg_out, deg_out, srcc, dstc, rowsv, aggsh, sem,
       onesv, zdeg, degsh) = rest
    else:
      agg_out, srcc, dstc, rowsv, aggsh, sem = rest
    c = lax.axis_index("c")
    s = lax.axis_index("s")
    ebase = s * T * CHUNK

    kpr = hd // LANES
    def zrow(i, _):
      rowsv[i // kpr, pl.ds((i % kpr) * LANES, LANES)] = jnp.zeros(
          (LANES,), jnp.float32)
      return 0
    lax.fori_loop(0, CHUNK * kpr, zrow, 0)
    base = s * orow
    nfull = orow // CHUNK
    for k in range(nfull):
      pltpu.sync_copy(rowsv, aggsh.at[pl.ds(base + k * CHUNK, CHUNK)])
    rem = orow - nfull * CHUNK
    if rem:
      pltpu.sync_copy(rowsv.at[pl.ds(0, rem)],
                      aggsh.at[pl.ds(base + nfull * CHUNK, rem)])
    if with_deg:
      def fill(i, _):
        onesv[i, :] = jnp.ones((LANES,), jnp.float32)
        zdeg[i, :] = jnp.zeros((LANES,), jnp.float32)
        return 0
      lax.fori_loop(0, CHUNK, fill, 0)
      @pl.when(c == 0)
      def _():
        for k in range(nfull):
          pltpu.sync_copy(zdeg, degsh.at[pl.ds(base + k * CHUNK, CHUNK)])
        if rem:
          pltpu.sync_copy(zdeg.at[pl.ds(0, rem)],
                          degsh.at[pl.ds(base + nfull * CHUNK, rem)])

    plsc.subcore_barrier()

    def step(j, _):
      eoff = ebase + j * CHUNK
      pltpu.sync_copy(src_hbm.at[pl.ds(eoff, CHUNK)], srcc)
      pltpu.sync_copy(dst_hbm.at[pl.ds(eoff, CHUNK)], dstc)
      pltpu.async_copy(x_hbm.at[c].at[srcc], rowsv, sem).wait()
      pltpu.sync_copy(rowsv, aggsh.at[dstc], add=True)
      if with_deg:
        @pl.when(c == 0)
        def _():
          pltpu.sync_copy(onesv, degsh.at[dstc], add=True)
      return 0
    lax.fori_loop(0, T, step, 0)

    plsc.subcore_barrier()

    pltpu.sync_copy(aggsh.at[pl.ds(base, orow)],
                    agg_out.at[c, pl.ds(base, orow)])
    if with_deg:
      @pl.when(c == 0)
      def _():
        pltpu.sync_copy(degsh.at[pl.ds(base, orow)],
                        deg_out.at[pl.ds(base, orow)])

  fn = pl.kernel(body, out_type=tuple(outs), mesh=_mesh(),
                 scratch_types=scratch,
                 compiler_params=pltpu.CompilerParams(
                     use_tc_tiling_on_sc=False))
  return fn(xh, src, dst)


def _dense(apart, deg, xh, wl, wr, b, relu, out_halves):
  _, n, hd = xh.shape
  d = 2 * hd
  h = wl.shape[1]
  R = 1000

  def body(ap_ref, dp_ref, x_ref, wl_ref, wr_ref, b_ref, o_ref):
    a = jnp.concatenate([ap_ref[0], ap_ref[1]], axis=-1)
    x = jnp.concatenate([x_ref[0], x_ref[1]], axis=-1)
    deg = jnp.maximum(dp_ref[:, 0:1], 1.0)
    mean = a / deg
    o = (jnp.dot(mean, wl_ref[...], preferred_element_type=jnp.float32)
         + jnp.dot(x, wr_ref[...], preferred_element_type=jnp.float32)
         + b_ref[...])
    o = jnp.maximum(o, 0.0) if relu else o
    if out_halves:
      o_ref[0] = o[:, :h // 2]
      o_ref[1] = o[:, h // 2:]
    else:
      o_ref[...] = o

  if out_halves:
    out_shape = jax.ShapeDtypeStruct((NC, n, h // 2), jnp.float32)
    out_specs = pl.BlockSpec((NC, R, h // 2), lambda i: (0, i, 0))
  else:
    out_shape = jax.ShapeDtypeStruct((n, h), jnp.float32)
    out_specs = pl.BlockSpec((R, h), lambda i: (i, 0))

  return pl.pallas_call(
      body,
      grid=(n // R,),
      in_specs=[
          pl.BlockSpec((NC, R, hd), lambda i: (0, i, 0)),
          pl.BlockSpec((R, LANES), lambda i: (i, 0)),
          pl.BlockSpec((NC, R, hd), lambda i: (0, i, 0)),
          pl.BlockSpec((d, h), lambda i: (0, 0)),
          pl.BlockSpec((d, h), lambda i: (0, 0)),
          pl.BlockSpec((1, h), lambda i: (0, 0)),
      ],
      out_specs=out_specs,
      out_shape=out_shape,
  )(apart, deg, xh, wl, wr, b.reshape(1, h))


def _score(z, ps, pd):
  n, d = z.shape
  ep = ps.shape[0]
  T = ep // (NW * CHUNK)

  def body(z_hbm, ps_hbm, pd_hbm, out_hbm, psc, pdc, av, bv, resv, sem):
    c = lax.axis_index("c")
    s = lax.axis_index("s")
    ebase = (c * NS + s) * T * CHUNK
    lanes_iota = lax.iota(jnp.int32, LANES)

    def step(j, _):
      eoff = ebase + j * CHUNK
      pltpu.sync_copy(ps_hbm.at[pl.ds(eoff, CHUNK)], psc)
      pltpu.sync_copy(pd_hbm.at[pl.ds(eoff, CHUNK)], pdc)
      pltpu.async_copy(z_hbm.at[psc], av, sem).wait()
      pltpu.async_copy(z_hbm.at[pdc], bv, sem).wait()
      def group(g, _):
        def edge(r, vec):
          row = g * LANES + r
          acc = av[row, pl.ds(0, LANES)] * bv[row, pl.ds(0, LANES)]
          for k in range(1, d // LANES):
            acc = acc + (av[row, pl.ds(k * LANES, LANES)]
                         * bv[row, pl.ds(k * LANES, LANES)])
          return jnp.where(lanes_iota == r, jnp.sum(acc), vec)
        vec = lax.fori_loop(0, LANES, edge, jnp.zeros((LANES,), jnp.float32))
        resv[pl.ds(g * LANES, LANES)] = vec
        return 0
      lax.fori_loop(0, CHUNK // LANES, group, 0)
      pltpu.sync_copy(resv, out_hbm.at[pl.ds(eoff, CHUNK)])
      return 0
    lax.fori_loop(0, T, step, 0)

  fn = pl.kernel(
      body,
      out_type=jax.ShapeDtypeStruct((ep,), jnp.float32),
      mesh=_mesh(),
      scratch_types=[
          pltpu.VMEM((CHUNK,), jnp.int32),
          pltpu.VMEM((CHUNK,), jnp.int32),
          pltpu.VMEM((CHUNK, d), jnp.float32),
          pltpu.VMEM((CHUNK, d), jnp.float32),
          pltpu.VMEM((CHUNK,), jnp.float32),
          pltpu.SemaphoreType.DMA,
      ],
      compiler_params=pltpu.CompilerParams(needs_layout_passes=False))
  return fn(z, ps, pd)


def _padflat(idx, e, ep, padval):
  return jnp.concatenate([idx, jnp.full((ep - e,), padval, jnp.int32)])


def kernel(x, edge_index, pred_edges, W1l, b1, W1r, W2l, b2, W2r):
  n, d = x.shape
  e = edge_index.shape[1]
  T = -(-e // (NW * CHUNK))
  ep = NW * T * CHUNK
  hd = d // 2

  src = _padflat(edge_index[0], e, ep, 0)
  dst = _padflat(edge_index[1], e, ep, n)

  xh = jnp.stack([x[:, :hd], x[:, hd:]])
  apart, deg = _agg(xh, src, dst, with_deg=True)
  hh = _dense(apart, deg, xh, W1l, W1r, b1, relu=True, out_halves=True)
  apart2, = _agg(hh, src, dst, with_deg=False)
  z = _dense(apart2, deg, hh, W2l, W2r, b2, relu=False, out_halves=False)

  ps = _padflat(pred_edges[0], e, ep, 0)
  pd = _padflat(pred_edges[1], e, ep, 0)
  scores = _score(z, ps, pd)
  return scores[:e]


# speedup vs baseline: 4.1831x; 1.5078x over previous
import jax
import jax.numpy as jnp
from jax import lax
from jax.experimental import pallas as pl
from jax.experimental.pallas import tpu as pltpu
from jax.experimental.pallas import tpu_sc as plsc

NC = 2
NS = 16
NW = NC * NS
LANES = 16
CHUNK = 128


def _mesh():
  return plsc.VectorSubcoreMesh(core_axis_name="c", subcore_axis_name="s",
                                num_cores=NC, num_subcores=NS)


def _agg(xh, e2, n, with_deg):
  _, _, hd = xh.shape
  nch = e2.shape[0]
  T = nch // NS
  M = T // 2
  assert T % 2 == 0
  orow = (-(-n // NS) + 7) // 8 * 8
  npad = orow * NS

  outs = [jax.ShapeDtypeStruct((NC, npad, hd), jnp.float32)]
  scratch = [
      pltpu.VMEM((2, CHUNK), jnp.int32),
      pltpu.VMEM((2, CHUNK), jnp.int32),
      pltpu.VMEM((CHUNK, hd), jnp.float32),
      pltpu.VMEM((CHUNK, hd), jnp.float32),
      pltpu.VMEM_SHARED((npad, hd), jnp.float32),
      pltpu.SemaphoreType.DMA,
      pltpu.SemaphoreType.DMA,
      pltpu.SemaphoreType.DMA,
      pltpu.SemaphoreType.DMA,
  ]
  if with_deg:
    outs.append(jax.ShapeDtypeStruct((npad, LANES), jnp.float32))
    scratch += [
        pltpu.VMEM((CHUNK, LANES), jnp.float32),
        pltpu.VMEM((CHUNK, LANES), jnp.float32),
        pltpu.VMEM_SHARED((npad, LANES), jnp.float32),
    ]

  def body(x_hbm, e2_hbm, *rest):
    if with_deg:
      (agg_out, deg_out, idxb0, idxb1, rows0, rows1, aggsh,
       sg0, sg1, si0, si1, onesv, zdeg, degsh) = rest
    else:
      (agg_out, idxb0, idxb1, rows0, rows1, aggsh,
       sg0, sg1, si0, si1) = rest
    c = lax.axis_index("c")
    s = lax.axis_index("s")
    x2 = x_hbm.at[c]
    cb = s * T

    kpr = hd // LANES
    def zrow(i, _):
      rows0[i // kpr, pl.ds((i % kpr) * LANES, LANES)] = jnp.zeros(
          (LANES,), jnp.float32)
      return 0
    lax.fori_loop(0, CHUNK * kpr, zrow, 0)
    base = s * orow
    nfull = orow // CHUNK
    for k in range(nfull):
      pltpu.sync_copy(rows0, aggsh.at[pl.ds(base + k * CHUNK, CHUNK)])
    rem = orow - nfull * CHUNK
    if rem:
      pltpu.sync_copy(rows0.at[pl.ds(0, rem)],
                      aggsh.at[pl.ds(base + nfull * CHUNK, rem)])
    if with_deg:
      def fill(i, _):
        onesv[i, :] = jnp.ones((LANES,), jnp.float32)
        zdeg[i, :] = jnp.zeros((LANES,), jnp.float32)
        return 0
      lax.fori_loop(0, CHUNK, fill, 0)
      @pl.when(c == 0)
      def _():
        for k in range(nfull):
          pltpu.sync_copy(zdeg, degsh.at[pl.ds(base + k * CHUNK, CHUNK)])
        if rem:
          pltpu.sync_copy(zdeg.at[pl.ds(0, rem)],
                          degsh.at[pl.ds(base + nfull * CHUNK, rem)])

    plsc.subcore_barrier()

    def scatter(rowsb, idxb):
      pltpu.sync_copy(rowsb, aggsh.at[idxb.at[1]], add=True)
      if with_deg:
        @pl.when(c == 0)
        def _():
          pltpu.sync_copy(onesv, degsh.at[idxb.at[1]], add=True)

    pltpu.sync_copy(e2_hbm.at[cb], idxb0)
    pltpu.async_copy(x2.at[idxb0.at[0]], rows0, sg0)
    pltpu.async_copy(e2_hbm.at[cb + 1], idxb1, si1)

    def step2(m, _):
      j0 = cb + 2 * m
      pltpu.make_async_copy(e2_hbm.at[j0 + 1], idxb1, si1).wait()
      pltpu.async_copy(x2.at[idxb1.at[0]], rows1, sg1)
      pltpu.make_async_copy(x2.at[pl.ds(0, CHUNK)], rows0, sg0).wait()
      scatter(rows0, idxb0)
      @pl.when(m + 1 < M)
      def _():
        pltpu.async_copy(e2_hbm.at[j0 + 2], idxb0, si0)
      pltpu.make_async_copy(x2.at[pl.ds(0, CHUNK)], rows1, sg1).wait()
      scatter(rows1, idxb1)
      @pl.when(m + 1 < M)
      def _():
        pltpu.make_async_copy(e2_hbm.at[j0 + 2], idxb0, si0).wait()
        pltpu.async_copy(x2.at[idxb0.at[0]], rows0, sg0)
        pltpu.async_copy(e2_hbm.at[j0 + 3], idxb1, si1)
      return 0
    lax.fori_loop(0, M, step2, 0)

    plsc.subcore_barrier()

    pltpu.sync_copy(aggsh.at[pl.ds(base, orow)],
                    agg_out.at[c, pl.ds(base, orow)])
    if with_deg:
      @pl.when(c == 0)
      def _():
        pltpu.sync_copy(degsh.at[pl.ds(base, orow)],
                        deg_out.at[pl.ds(base, orow)])

  fn = pl.kernel(body, out_type=tuple(outs), mesh=_mesh(),
                 scratch_types=scratch,
                 compiler_params=pltpu.CompilerParams(
                     use_tc_tiling_on_sc=False))
  return fn(xh, e2)


def _dense(apart, deg, xh, wl, wr, b, relu, out_halves):
  _, n, hd = xh.shape
  d = 2 * hd
  h = wl.shape[1]
  R = 1000

  def body(ap_ref, dp_ref, x_ref, wl_ref, wr_ref, b_ref, o_ref):
    a = jnp.concatenate([ap_ref[0], ap_ref[1]], axis=-1)
    x = jnp.concatenate([x_ref[0], x_ref[1]], axis=-1)
    deg = jnp.maximum(dp_ref[:, 0:1], 1.0)
    mean = a / deg
    o = (jnp.dot(mean, wl_ref[...], preferred_element_type=jnp.float32)
         + jnp.dot(x, wr_ref[...], preferred_element_type=jnp.float32)
         + b_ref[...])
    o = jnp.maximum(o, 0.0) if relu else o
    if out_halves:
      o_ref[0] = o[:, :h // 2]
      o_ref[1] = o[:, h // 2:]
    else:
      o_ref[...] = o

  if out_halves:
    out_shape = jax.ShapeDtypeStruct((NC, n, h // 2), jnp.float32)
    out_specs = pl.BlockSpec((NC, R, h // 2), lambda i: (0, i, 0))
  else:
    out_shape = jax.ShapeDtypeStruct((n, h), jnp.float32)
    out_specs = pl.BlockSpec((R, h), lambda i: (i, 0))

  return pl.pallas_call(
      body,
      grid=(n // R,),
      in_specs=[
          pl.BlockSpec((NC, R, hd), lambda i: (0, i, 0)),
          pl.BlockSpec((R, LANES), lambda i: (i, 0)),
          pl.BlockSpec((NC, R, hd), lambda i: (0, i, 0)),
          pl.BlockSpec((d, h), lambda i: (0, 0)),
          pl.BlockSpec((d, h), lambda i: (0, 0)),
          pl.BlockSpec((1, h), lambda i: (0, 0)),
      ],
      out_specs=out_specs,
      out_shape=out_shape,
  )(apart, deg, xh, wl, wr, b.reshape(1, h))


def _score(z, pe2):
  n, d = z.shape
  nch = pe2.shape[0]
  T = nch // NW
  M = (T - 1) // 2
  assert T % 2 == 1

  def body(z_hbm, pe2_hbm, out_hbm, idxall, av0, bv0, av1, bv1, resall,
           sg0, sg1):
    c = lax.axis_index("c")
    s = lax.axis_index("s")
    cb = (c * NS + s) * T
    lanes_iota = lax.iota(jnp.int32, LANES)

    pltpu.sync_copy(pe2_hbm.at[pl.ds(cb, T)], idxall)

    def gathers(j, av, bv, sg):
      pltpu.async_copy(z_hbm.at[idxall.at[j, 0]], av, sg)
      pltpu.async_copy(z_hbm.at[idxall.at[j, 1]], bv, sg)

    def drain2(av, bv, sg):
      pltpu.make_async_copy(z_hbm.at[pl.ds(0, CHUNK)], av, sg).wait()
      pltpu.make_async_copy(z_hbm.at[pl.ds(0, CHUNK)], bv, sg).wait()

    def compute(jl, av, bv):
      def group(g, _):
        def edge(r, vec):
          row = g * LANES + r
          acc = av[row, pl.ds(0, LANES)] * bv[row, pl.ds(0, LANES)]
          for k in range(1, d // LANES):
            acc = acc + (av[row, pl.ds(k * LANES, LANES)]
                         * bv[row, pl.ds(k * LANES, LANES)])
          return jnp.where(lanes_iota == r, jnp.sum(acc), vec)
        vec = lax.fori_loop(0, LANES, edge, jnp.zeros((LANES,), jnp.float32))
        resall[jl, pl.ds(g * LANES, LANES)] = vec
        return 0
      lax.fori_loop(0, CHUNK // LANES, group, 0)

    gathers(0, av0, bv0, sg0)
    gathers(1, av1, bv1, sg1)

    def step2(m, _):
      j0 = 2 * m
      drain2(av0, bv0, sg0)
      compute(j0, av0, bv0)
      gathers(j0 + 2, av0, bv0, sg0)
      drain2(av1, bv1, sg1)
      compute(j0 + 1, av1, bv1)
      @pl.when(m + 1 < M)
      def _():
        gathers(j0 + 3, av1, bv1, sg1)
      return 0
    lax.fori_loop(0, M, step2, 0)

    drain2(av0, bv0, sg0)
    compute(T - 1, av0, bv0)

    pltpu.sync_copy(resall, out_hbm.at[pl.ds(cb, T)])

  fn = pl.kernel(
      body,
      out_type=jax.ShapeDtypeStruct((nch, CHUNK), jnp.float32),
      mesh=_mesh(),
      scratch_types=[
          pltpu.VMEM((T, 2, CHUNK), jnp.int32),
          pltpu.VMEM((CHUNK, d), jnp.float32),
          pltpu.VMEM((CHUNK, d), jnp.float32),
          pltpu.VMEM((CHUNK, d), jnp.float32),
          pltpu.VMEM((CHUNK, d), jnp.float32),
          pltpu.VMEM((T, CHUNK), jnp.float32),
          pltpu.SemaphoreType.DMA,
          pltpu.SemaphoreType.DMA,
      ],
      compiler_params=pltpu.CompilerParams(needs_layout_passes=False,
                                           use_tc_tiling_on_sc=False))
  return fn(z, pe2)


def _chunk2(a, b, e, ep, apad, bpad):
  a = jnp.concatenate([a, jnp.full((ep - e,), apad, jnp.int32)])
  b = jnp.concatenate([b, jnp.full((ep - e,), bpad, jnp.int32)])
  return jnp.stack([a.reshape(-1, CHUNK), b.reshape(-1, CHUNK)], axis=1)


def kernel(x, edge_index, pred_edges, W1l, b1, W1r, W2l, b2, W2r):
  n, d = x.shape
  e = edge_index.shape[1]
  T = -(-e // (NW * CHUNK))
  ep = NW * T * CHUNK
  hd = d // 2

  e2 = _chunk2(edge_index[0], edge_index[1], e, ep, 0, n)

  xh = jnp.stack([x[:, :hd], x[:, hd:]])
  apart, deg = _agg(xh, e2, n, with_deg=True)
  hh = _dense(apart, deg, xh, W1l, W1r, b1, relu=True, out_halves=True)
  apart2, = _agg(hh, e2, n, with_deg=False)
  z = _dense(apart2, deg, hh, W2l, W2r, b2, relu=False, out_halves=False)

  pe2 = _chunk2(pred_edges[0], pred_edges[1], e, ep, 0, 0)
  scores = _score(z, pe2)
  return scores.reshape(-1)[:e]
